# baseline (speedup 1.0000x reference)
import functools
import math

import jax
import jax.numpy as jnp
import numpy as np
from jax import lax
from jax.experimental import pallas as pl
from jax.experimental.pallas import tpu as pltpu
from jax.experimental.pallas import tpu_sc as plsc

_N = 16384
_M = 4096
_K = 32
_CIN = 128
_COUT = 256
_DPAD = 144
_RB = 64
_NBLK = _M // _RB
_NCH = 128
_CS = _N // _NCH
_JMAX = 6
_NW = 32
_CH = 128
_BIG = 2**30

_sidx_np = None


def _sampled_idx():
    global _sidx_np
    if _sidx_np is None:
        try:
            with jax.ensure_compile_time_eval():
                _sidx_np = np.asarray(jax.random.permutation(jax.random.key(42), _N))[:_M]
        except Exception:
            return jax.random.permutation(jax.random.key(42), _N)[:_M]
    return jnp.asarray(_sidx_np)


def _topk_body(pos_t_ref, posd_ref, idx_ref, ups_ref, runmin_ref, runarg_ref):
    blk = pl.program_id(0)
    pn2 = pos_t_ref[5:6, :]
    pd = posd_ref[:]
    pd2 = pd[:, 4:5]
    dot = jax.lax.dot_general(
        pd.astype(jnp.bfloat16), pos_t_ref[:].astype(jnp.bfloat16),
        (((1,), (0,)), ((), ())), preferred_element_type=jnp.float32)
    d2 = (pd2 + pn2) - 2.0 * dot
    d2 = jnp.sqrt(jnp.maximum(d2, 0.0))

    colmin = jnp.min(d2, axis=0, keepdims=True)
    rowio = lax.broadcasted_iota(jnp.int32, d2.shape, 0) + blk * _RB
    colarg = jnp.min(jnp.where(d2 == colmin, rowio, _BIG), axis=0, keepdims=True)

    @pl.when(blk == 0)
    def _():
        runmin_ref[:] = colmin
        runarg_ref[:] = colarg

    @pl.when(blk > 0)
    def _():
        better = colmin < runmin_ref[:]
        runarg_ref[:] = jnp.where(better, colarg, runarg_ref[:])
        runmin_ref[:] = jnp.minimum(colmin, runmin_ref[:])

    @pl.when(blk == _NBLK - 1)
    def _():
        ups_ref[:] = runarg_ref[:]

    inf = jnp.float32(jnp.inf)
    kio = lax.broadcasted_iota(jnp.int32, (_RB, _K), 1)

    d3 = d2.reshape(_RB, _CS, _NCH)
    sio = lax.broadcasted_iota(jnp.int32, (_RB, _CS, _NCH), 1)
    cio = lax.broadcasted_iota(jnp.int32, (_RB, _NCH), 1)
    vs, js = [], []
    for _ in range(_JMAX):
        m = jnp.min(d3, axis=1, keepdims=True)
        am = jnp.min(jnp.where(d3 == m, sio, _BIG), axis=1, keepdims=True)
        d3 = jnp.where(sio == am, inf, d3)
        vs.append(m.reshape(_RB, _NCH))
        js.append(am.reshape(_RB, _NCH) * _NCH + cio)
    v2 = jnp.concatenate(vs, axis=1)
    i2 = jnp.concatenate(js, axis=1)

    floor = jnp.min(vs[-1], axis=1, keepdims=True)
    cnt = jnp.sum((v2 < floor).astype(jnp.int32), axis=1, keepdims=True)
    allsafe = jnp.all(cnt >= _K)

    def extract32(vals, idxs):
        def step(t, carry):
            v, acc = carry
            m = jnp.min(v, axis=1, keepdims=True)
            isel = jnp.min(jnp.where(v == m, idxs, _BIG), axis=1, keepdims=True)
            acc = jnp.where(kio == t, isel, acc)
            v = jnp.where((v == m) & (idxs == isel), inf, v)
            return v, acc

        acc0 = jnp.zeros((_RB, _K), jnp.int32)
        return lax.fori_loop(0, _K, step, (vals, acc0))[1]

    idx_ref[:] = lax.cond(
        allsafe,
        lambda: extract32(v2, i2),
        lambda: extract32(d2, lax.broadcasted_iota(jnp.int32, d2.shape, 1)),
    )


def _topk_call(pos_t8, posd_pad):
    return pl.pallas_call(
        _topk_body,
        grid=(_NBLK,),
        in_specs=[
            pl.BlockSpec((8, _N), lambda b: (0, 0)),
            pl.BlockSpec((_RB, 8), lambda b: (b, 0)),
        ],
        out_specs=[
            pl.BlockSpec((_RB, _K), lambda b: (b, 0)),
            pl.BlockSpec((1, _N), lambda b: (0, 0)),
        ],
        out_shape=[
            jax.ShapeDtypeStruct((_M, _K), jnp.int32),
            jax.ShapeDtypeStruct((1, _N), jnp.int32),
        ],
        scratch_shapes=[
            pltpu.VMEM((1, _N), jnp.float32),
            pltpu.VMEM((1, _N), jnp.int32),
        ],
    )(pos_t8, posd_pad)


def _premul_body(xp_ref, w1_ref, out_ref):
    out_ref[:] = jnp.dot(xp_ref[:], w1_ref[:], preferred_element_type=jnp.float32)


def _premul_call(xp, w1cat):
    return pl.pallas_call(
        _premul_body,
        grid=(4,),
        in_specs=[
            pl.BlockSpec((_N // 4, _DPAD), lambda b: (b, 0)),
            pl.BlockSpec((_DPAD, _COUT), lambda b: (0, 0)),
        ],
        out_specs=pl.BlockSpec((_N // 4, _COUT), lambda b: (b, 0)),
        out_shape=jax.ShapeDtypeStruct((_N, _COUT), jnp.float32),
    )(xp, w1cat)


def _make_sc_gather():
    b_per_w = _M * _K // _NW
    nch = b_per_w // _CH
    mesh = plsc.VectorSubcoreMesh(core_axis_name="c", subcore_axis_name="s")

    @functools.partial(
        pl.kernel,
        mesh=mesh,
        out_type=jax.ShapeDtypeStruct((_M * _K, _COUT), jnp.float32),
        scratch_types=[
            pltpu.VMEM((nch, _CH), jnp.int32),
            pltpu.VMEM((2, _CH, _COUT), jnp.float32),
            pltpu.SemaphoreType.DMA((2,)),
        ],
    )
    def sc_gather(table_hbm, idx_hbm, out_hbm, idx_v, rows_v, sems):
        wid = lax.axis_index("s") * 2 + lax.axis_index("c")
        base = wid * b_per_w
        pltpu.sync_copy(idx_hbm.at[wid], idx_v)
        pltpu.async_copy(table_hbm.at[idx_v.at[0]], rows_v.at[0], sems.at[0])

        def body(c, _):
            nxt = c + 1

            @pl.when(nxt < nch)
            def _():
                pltpu.async_copy(
                    table_hbm.at[idx_v.at[nxt]], rows_v.at[nxt % 2], sems.at[nxt % 2]
                )

            pltpu.make_async_copy(
                table_hbm.at[idx_v.at[c]], rows_v.at[c % 2], sems.at[c % 2]
            ).wait()
            pltpu.sync_copy(rows_v.at[c % 2], out_hbm.at[pl.ds(base + c * _CH, _CH)])
            return 0

        lax.fori_loop(0, nch, body, 0)

    return sc_gather


def _ln(h, g, b):
    mu = jnp.mean(h, axis=-1, keepdims=True)
    d = h - mu
    var = jnp.mean(d * d, axis=-1, keepdims=True)
    return d / jnp.sqrt(var + 1e-5) * g + b


def _gelu(h):
    return 0.5 * h * (1.0 + lax.erf(h * (1.0 / math.sqrt(2.0))))


def _mlp_body(g_ref, posd_ref, w1p_ref, b1_ref, g1_ref, be1_ref,
              w2_ref, b2_ref, g2_ref, be2_ref, out_ref, pdw_ref, mx_ref):
    g = pl.program_id(0)

    @pl.when(g == 0)
    def _():
        pdw_ref[:] = jnp.dot(
            posd_ref[:], w1p_ref[:], preferred_element_type=jnp.float32) - b1_ref[:]

    h = g_ref[:] - pdw_ref[:]
    h = _gelu(_ln(h, g1_ref[:], be1_ref[:]))
    h = jnp.dot(h, w2_ref[:], preferred_element_type=jnp.float32) + b2_ref[:]
    h = _gelu(_ln(h, g2_ref[:], be2_ref[:]))

    @pl.when(g == 0)
    def _():
        mx_ref[:] = h

    @pl.when(g > 0)
    def _():
        mx_ref[:] = jnp.maximum(mx_ref[:], h)

    @pl.when(g == _K - 1)
    def _():
        out_ref[:] = mx_ref[:]


def _mlp_call(gathered, posd_pad, w1pos, b1, g1, be1, w2, b2, g2, be2):
    full = lambda r, c: pl.BlockSpec((r, c), lambda g: (0, 0))
    return pl.pallas_call(
        _mlp_body,
        grid=(_K,),
        in_specs=[
            pl.BlockSpec((_M, _COUT), lambda g: (g, 0)),
            full(_M, 8),
            full(8, _COUT),
            full(1, _COUT),
            full(1, _COUT),
            full(1, _COUT),
            full(_COUT, _COUT),
            full(1, _COUT),
            full(1, _COUT),
            full(1, _COUT),
        ],
        out_specs=pl.BlockSpec((_M, _COUT), lambda g: (0, 0)),
        out_shape=jax.ShapeDtypeStruct((_M, _COUT), jnp.float32),
        scratch_shapes=[
            pltpu.VMEM((_M, _COUT), jnp.float32),
            pltpu.VMEM((_M, _COUT), jnp.float32),
        ],
    )(gathered, posd_pad, w1pos, b1, g1, be1, w2, b2, g2, be2)


def kernel(x, pos, batch, W1, b1, g1, beta1, W2, b2, g2, beta2):
    sidx = _sampled_idx()
    pos_down = pos[sidx]
    batch_down = batch[sidx]

    pd2 = jnp.sum(pos_down * pos_down, axis=1).reshape(_M, 1)
    pn2 = jnp.sum(pos * pos, axis=1).reshape(1, _N)
    z1 = jnp.zeros((_M, 1), jnp.float32)
    posd_pad = jnp.concatenate([pos_down, z1, pd2, z1, z1, z1], axis=1)
    zr = jnp.zeros((1, _N), jnp.float32)
    pos_t8 = jnp.concatenate([pos.T, zr, zr, pn2, zr, zr], axis=0)

    knn_idx, ups2d = _topk_call(pos_t8, posd_pad)
    upsample_idx = ups2d.reshape(_N)

    idx_kmajor = knn_idx.T.reshape(_NW, _M * _K // _NW // _CH, _CH)

    xp = jnp.concatenate([x, pos, jnp.zeros((_N, _DPAD - _CIN - 3), jnp.float32)], axis=1)
    w1cat = jnp.concatenate([W1, jnp.zeros((_DPAD - _CIN - 3, _COUT), jnp.float32)], axis=0)
    table = _premul_call(xp, w1cat)
    gathered = _make_sc_gather()(table, idx_kmajor)

    w1pos = jnp.concatenate([W1[_CIN:_CIN + 3], jnp.zeros((5, _COUT), jnp.float32)], axis=0)
    r = lambda v: v.reshape(1, _COUT)

    x_down = _mlp_call(gathered, posd_pad, w1pos,
                       r(b1), r(g1), r(beta1), W2, r(b2), r(g2), r(beta2))
    return (x_down, pos_down, batch_down, upsample_idx)


# speedup vs baseline: 1.1522x; 1.1522x over previous
import functools
import math

import jax
import jax.numpy as jnp
import numpy as np
from jax import lax
from jax.experimental import pallas as pl
from jax.experimental.pallas import tpu as pltpu
from jax.experimental.pallas import tpu_sc as plsc

_N = 16384
_M = 4096
_K = 32
_CIN = 128
_COUT = 256
_DPAD = 144
_RB = 128
_NBLK = _M // _RB
_NCH = 128
_CS = _N // _NCH
_JMAX = 6
_NW = 32
_CH = 128
_BIG = 2**30

_sidx_np = None


def _sampled_idx():
    global _sidx_np
    if _sidx_np is None:
        try:
            with jax.ensure_compile_time_eval():
                _sidx_np = np.asarray(jax.random.permutation(jax.random.key(42), _N))[:_M]
        except Exception:
            return jax.random.permutation(jax.random.key(42), _N)[:_M]
    return jnp.asarray(_sidx_np)


def _topk_body(pos_t_ref, posd_ref, idx_ref, ups_ref, runmin_ref, runarg_ref):
    blk = pl.program_id(0)
    pn2 = pos_t_ref[5:6, :]
    pd = posd_ref[:]
    pd2 = pd[:, 4:5]
    dot = jax.lax.dot_general(
        pd.astype(jnp.bfloat16), pos_t_ref[:].astype(jnp.bfloat16),
        (((1,), (0,)), ((), ())), preferred_element_type=jnp.float32)
    d2 = (pd2 + pn2) - 2.0 * dot
    d2 = jnp.sqrt(jnp.maximum(d2, 0.0))

    colmin = jnp.min(d2, axis=0, keepdims=True)
    rowio = lax.broadcasted_iota(jnp.int32, d2.shape, 0) + blk * _RB
    colarg = jnp.min(jnp.where(d2 == colmin, rowio, _BIG), axis=0, keepdims=True)

    @pl.when(blk == 0)
    def _():
        runmin_ref[:] = colmin
        runarg_ref[:] = colarg

    @pl.when(blk > 0)
    def _():
        better = colmin < runmin_ref[:]
        runarg_ref[:] = jnp.where(better, colarg, runarg_ref[:])
        runmin_ref[:] = jnp.minimum(colmin, runmin_ref[:])

    @pl.when(blk == _NBLK - 1)
    def _():
        ups_ref[:] = runarg_ref[:]

    inf = jnp.float32(jnp.inf)
    kio = lax.broadcasted_iota(jnp.int32, (_RB, _K), 1)

    d3 = d2.reshape(_RB, _CS, _NCH)
    sio = lax.broadcasted_iota(jnp.int32, (_RB, _CS, _NCH), 1)
    cio = lax.broadcasted_iota(jnp.int32, (_RB, _NCH), 1)
    vs, js = [], []
    for _ in range(_JMAX):
        m = jnp.min(d3, axis=1, keepdims=True)
        am = jnp.argmin(d3, axis=1).astype(jnp.int32).reshape(_RB, 1, _NCH)
        d3 = jnp.where(sio == am, inf, d3)
        vs.append(m.reshape(_RB, _NCH))
        js.append(am.reshape(_RB, _NCH) * _NCH + cio)
    v2 = jnp.concatenate(vs, axis=1)
    i2 = jnp.concatenate(js, axis=1)

    floor = jnp.min(vs[-1], axis=1, keepdims=True)
    cnt = jnp.sum((v2 < floor).astype(jnp.int32), axis=1, keepdims=True)
    allsafe = jnp.all(cnt >= _K)

    def extract32(vals, idxs):
        def step(t, carry):
            v, acc = carry
            m = jnp.min(v, axis=1, keepdims=True)
            isel = jnp.min(jnp.where(v == m, idxs, _BIG), axis=1, keepdims=True)
            acc = jnp.where(kio == t, isel, acc)
            v = jnp.where((v == m) & (idxs == isel), inf, v)
            return v, acc

        acc0 = jnp.zeros((_RB, _K), jnp.int32)
        return lax.fori_loop(0, _K, step, (vals, acc0))[1]

    idx_ref[:] = lax.cond(
        allsafe,
        lambda: extract32(v2, i2),
        lambda: extract32(d2, lax.broadcasted_iota(jnp.int32, d2.shape, 1)),
    )


def _topk_call(pos_t8, posd_pad):
    return pl.pallas_call(
        _topk_body,
        grid=(_NBLK,),
        in_specs=[
            pl.BlockSpec((8, _N), lambda b: (0, 0)),
            pl.BlockSpec((_RB, 8), lambda b: (b, 0)),
        ],
        out_specs=[
            pl.BlockSpec((_RB, _K), lambda b: (b, 0)),
            pl.BlockSpec((1, _N), lambda b: (0, 0)),
        ],
        out_shape=[
            jax.ShapeDtypeStruct((_M, _K), jnp.int32),
            jax.ShapeDtypeStruct((1, _N), jnp.int32),
        ],
        scratch_shapes=[
            pltpu.VMEM((1, _N), jnp.float32),
            pltpu.VMEM((1, _N), jnp.int32),
        ],
    )(pos_t8, posd_pad)


def _premul_body(xp_ref, w1_ref, out_ref):
    out_ref[:] = jnp.dot(xp_ref[:], w1_ref[:], preferred_element_type=jnp.float32)


def _premul_call(xp, w1cat):
    return pl.pallas_call(
        _premul_body,
        grid=(4,),
        in_specs=[
            pl.BlockSpec((_N // 4, _DPAD), lambda b: (b, 0)),
            pl.BlockSpec((_DPAD, _COUT), lambda b: (0, 0)),
        ],
        out_specs=pl.BlockSpec((_N // 4, _COUT), lambda b: (b, 0)),
        out_shape=jax.ShapeDtypeStruct((_N, _COUT), jnp.float32),
    )(xp, w1cat)


def _make_sc_gather():
    b_per_w = _M * _K // _NW
    nch = b_per_w // _CH
    mesh = plsc.VectorSubcoreMesh(core_axis_name="c", subcore_axis_name="s")

    @functools.partial(
        pl.kernel,
        mesh=mesh,
        out_type=jax.ShapeDtypeStruct((_M * _K, _COUT), jnp.float32),
        scratch_types=[
            pltpu.VMEM((nch, _CH), jnp.int32),
            pltpu.VMEM((2, _CH, _COUT), jnp.float32),
            pltpu.SemaphoreType.DMA((2,)),
        ],
    )
    def sc_gather(table_hbm, idx_hbm, out_hbm, idx_v, rows_v, sems):
        wid = lax.axis_index("s") * 2 + lax.axis_index("c")
        base = wid * b_per_w
        pltpu.sync_copy(idx_hbm.at[wid], idx_v)
        pltpu.async_copy(table_hbm.at[idx_v.at[0]], rows_v.at[0], sems.at[0])

        def body(c, _):
            nxt = c + 1

            @pl.when(nxt < nch)
            def _():
                pltpu.async_copy(
                    table_hbm.at[idx_v.at[nxt]], rows_v.at[nxt % 2], sems.at[nxt % 2]
                )

            pltpu.make_async_copy(
                table_hbm.at[idx_v.at[c]], rows_v.at[c % 2], sems.at[c % 2]
            ).wait()
            pltpu.sync_copy(rows_v.at[c % 2], out_hbm.at[pl.ds(base + c * _CH, _CH)])
            return 0

        lax.fori_loop(0, nch, body, 0)

    return sc_gather


def _ln(h, g, b):
    mu = jnp.mean(h, axis=-1, keepdims=True)
    d = h - mu
    var = jnp.mean(d * d, axis=-1, keepdims=True)
    return d / jnp.sqrt(var + 1e-5) * g + b


def _gelu(h):
    return 0.5 * h * (1.0 + lax.erf(h * (1.0 / math.sqrt(2.0))))


def _mlp_body(g_ref, posd_ref, w1p_ref, b1_ref, g1_ref, be1_ref,
              w2_ref, b2_ref, g2_ref, be2_ref, out_ref, pdw_ref, mx_ref):
    g = pl.program_id(0)

    @pl.when(g == 0)
    def _():
        pdw_ref[:] = jnp.dot(
            posd_ref[:], w1p_ref[:], preferred_element_type=jnp.float32) - b1_ref[:]

    h = g_ref[:] - pdw_ref[:]
    h = _gelu(_ln(h, g1_ref[:], be1_ref[:]))
    h = jnp.dot(h, w2_ref[:], preferred_element_type=jnp.float32) + b2_ref[:]
    h = _gelu(_ln(h, g2_ref[:], be2_ref[:]))

    @pl.when(g == 0)
    def _():
        mx_ref[:] = h

    @pl.when(g > 0)
    def _():
        mx_ref[:] = jnp.maximum(mx_ref[:], h)

    @pl.when(g == _K - 1)
    def _():
        out_ref[:] = mx_ref[:]


def _mlp_call(gathered, posd_pad, w1pos, b1, g1, be1, w2, b2, g2, be2):
    full = lambda r, c: pl.BlockSpec((r, c), lambda g: (0, 0))
    return pl.pallas_call(
        _mlp_body,
        grid=(_K,),
        in_specs=[
            pl.BlockSpec((_M, _COUT), lambda g: (g, 0)),
            full(_M, 8),
            full(8, _COUT),
            full(1, _COUT),
            full(1, _COUT),
            full(1, _COUT),
            full(_COUT, _COUT),
            full(1, _COUT),
            full(1, _COUT),
            full(1, _COUT),
        ],
        out_specs=pl.BlockSpec((_M, _COUT), lambda g: (0, 0)),
        out_shape=jax.ShapeDtypeStruct((_M, _COUT), jnp.float32),
        scratch_shapes=[
            pltpu.VMEM((_M, _COUT), jnp.float32),
            pltpu.VMEM((_M, _COUT), jnp.float32),
        ],
    )(gathered, posd_pad, w1pos, b1, g1, be1, w2, b2, g2, be2)


def kernel(x, pos, batch, W1, b1, g1, beta1, W2, b2, g2, beta2):
    sidx = _sampled_idx()
    pos_down = pos[sidx]
    batch_down = batch[sidx]

    pd2 = jnp.sum(pos_down * pos_down, axis=1).reshape(_M, 1)
    pn2 = jnp.sum(pos * pos, axis=1).reshape(1, _N)
    z1 = jnp.zeros((_M, 1), jnp.float32)
    posd_pad = jnp.concatenate([pos_down, z1, pd2, z1, z1, z1], axis=1)
    zr = jnp.zeros((1, _N), jnp.float32)
    pos_t8 = jnp.concatenate([pos.T, zr, zr, pn2, zr, zr], axis=0)

    knn_idx, ups2d = _topk_call(pos_t8, posd_pad)
    upsample_idx = ups2d.reshape(_N)

    idx_kmajor = knn_idx.T.reshape(_NW, _M * _K // _NW // _CH, _CH)

    xp = jnp.concatenate([x, pos, jnp.zeros((_N, _DPAD - _CIN - 3), jnp.float32)], axis=1)
    w1cat = jnp.concatenate([W1, jnp.zeros((_DPAD - _CIN - 3, _COUT), jnp.float32)], axis=0)
    table = _premul_call(xp, w1cat)
    gathered = _make_sc_gather()(table, idx_kmajor)

    w1pos = jnp.concatenate([W1[_CIN:_CIN + 3], jnp.zeros((5, _COUT), jnp.float32)], axis=0)
    r = lambda v: v.reshape(1, _COUT)

    x_down = _mlp_call(gathered, posd_pad, w1pos,
                       r(b1), r(g1), r(beta1), W2, r(b2), r(g2), r(beta2))
    return (x_down, pos_down, batch_down, upsample_idx)


# speedup vs baseline: 1.1530x; 1.0006x over previous
import functools
import math

import jax
import jax.numpy as jnp
import numpy as np
from jax import lax
from jax.experimental import pallas as pl
from jax.experimental.pallas import tpu as pltpu
from jax.experimental.pallas import tpu_sc as plsc

_N = 16384
_M = 4096
_K = 32
_CIN = 128
_COUT = 256
_DPAD = 144
_RB = 128
_NBLK = _M // _RB
_NCH = 128
_CS = _N // _NCH
_JMAX = 6
_NW = 32
_CH = 128
_BIG = 2**30

_sidx_np = None


def _sampled_idx():
    global _sidx_np
    if _sidx_np is None:
        try:
            with jax.ensure_compile_time_eval():
                _sidx_np = np.asarray(jax.random.permutation(jax.random.key(42), _N))[:_M]
        except Exception:
            return jax.random.permutation(jax.random.key(42), _N)[:_M]
    return jnp.asarray(_sidx_np)


def _topk_body(pos_t_ref, posd_ref, idx_ref, ups_ref, runmin_ref, runarg_ref):
    blk = pl.program_id(0)
    pn2 = pos_t_ref[5:6, :]
    pd = posd_ref[:]
    pd2 = pd[:, 4:5]
    dot = jax.lax.dot_general(
        pd.astype(jnp.bfloat16), pos_t_ref[:].astype(jnp.bfloat16),
        (((1,), (0,)), ((), ())), preferred_element_type=jnp.float32)
    d2 = (pd2 + pn2) - 2.0 * dot
    d2 = jnp.sqrt(jnp.maximum(d2, 0.0))

    colmin = jnp.min(d2, axis=0, keepdims=True)
    rowio = lax.broadcasted_iota(jnp.int32, d2.shape, 0) + blk * _RB
    colarg = jnp.min(jnp.where(d2 == colmin, rowio, _BIG), axis=0, keepdims=True)

    @pl.when(blk == 0)
    def _():
        runmin_ref[:] = colmin
        runarg_ref[:] = colarg

    @pl.when(blk > 0)
    def _():
        better = colmin < runmin_ref[:]
        runarg_ref[:] = jnp.where(better, colarg, runarg_ref[:])
        runmin_ref[:] = jnp.minimum(colmin, runmin_ref[:])

    @pl.when(blk == _NBLK - 1)
    def _():
        ups_ref[:] = runarg_ref[:]

    inf = jnp.float32(jnp.inf)
    kio = lax.broadcasted_iota(jnp.int32, (_RB, _K), 1)

    d3 = d2.reshape(_RB, _CS, _NCH)
    sio = lax.broadcasted_iota(jnp.int32, (_RB, _CS, _NCH), 1)
    cio = lax.broadcasted_iota(jnp.int32, (_RB, _NCH), 1)
    vs, js = [], []
    for _ in range(_JMAX):
        m = jnp.min(d3, axis=1, keepdims=True)
        am = jnp.min(jnp.where(d3 == m, sio, _BIG), axis=1, keepdims=True)
        d3 = jnp.where(sio == am, inf, d3)
        vs.append(m.reshape(_RB, _NCH))
        js.append(am.reshape(_RB, _NCH) * _NCH + cio)
    v2 = jnp.concatenate(vs, axis=1)
    i2 = jnp.concatenate(js, axis=1)

    floor = jnp.min(vs[-1], axis=1, keepdims=True)
    cnt = jnp.sum((v2 < floor).astype(jnp.int32), axis=1, keepdims=True)
    allsafe = jnp.all(cnt >= _K)

    def extract32(vals, idxs):
        def step(t, carry):
            v, acc = carry
            m = jnp.min(v, axis=1, keepdims=True)
            isel = jnp.min(jnp.where(v == m, idxs, _BIG), axis=1, keepdims=True)
            acc = jnp.where(kio == t, isel, acc)
            v = jnp.where((v == m) & (idxs == isel), inf, v)
            return v, acc

        acc0 = jnp.zeros((_RB, _K), jnp.int32)
        return lax.fori_loop(0, _K, step, (vals, acc0))[1]

    idx_ref[:] = lax.cond(
        allsafe,
        lambda: extract32(v2, i2),
        lambda: extract32(d2, lax.broadcasted_iota(jnp.int32, d2.shape, 1)),
    )


def _topk_call(pos_t8, posd_pad):
    return pl.pallas_call(
        _topk_body,
        grid=(_NBLK,),
        in_specs=[
            pl.BlockSpec((8, _N), lambda b: (0, 0)),
            pl.BlockSpec((_RB, 8), lambda b: (b, 0)),
        ],
        out_specs=[
            pl.BlockSpec((_RB, _K), lambda b: (b, 0)),
            pl.BlockSpec((1, _N), lambda b: (0, 0)),
        ],
        out_shape=[
            jax.ShapeDtypeStruct((_M, _K), jnp.int32),
            jax.ShapeDtypeStruct((1, _N), jnp.int32),
        ],
        scratch_shapes=[
            pltpu.VMEM((1, _N), jnp.float32),
            pltpu.VMEM((1, _N), jnp.int32),
        ],
    )(pos_t8, posd_pad)


def _premul_body(xp_ref, w1_ref, out_ref):
    out_ref[:] = jnp.dot(xp_ref[:], w1_ref[:], preferred_element_type=jnp.float32)


def _premul_call(xp, w1cat):
    return pl.pallas_call(
        _premul_body,
        grid=(4,),
        in_specs=[
            pl.BlockSpec((_N // 4, _DPAD), lambda b: (b, 0)),
            pl.BlockSpec((_DPAD, _COUT), lambda b: (0, 0)),
        ],
        out_specs=pl.BlockSpec((_N // 4, _COUT), lambda b: (b, 0)),
        out_shape=jax.ShapeDtypeStruct((_N, _COUT), jnp.float32),
    )(xp, w1cat)


def _make_sc_gather():
    b_per_w = _M * _K // _NW
    nch = b_per_w // _CH
    mesh = plsc.VectorSubcoreMesh(core_axis_name="c", subcore_axis_name="s")

    @functools.partial(
        pl.kernel,
        mesh=mesh,
        out_type=jax.ShapeDtypeStruct((_M * _K, _COUT), jnp.float32),
        scratch_types=[
            pltpu.VMEM((nch, _CH), jnp.int32),
            pltpu.VMEM((3, _CH, _COUT), jnp.float32),
            pltpu.SemaphoreType.DMA((3,)),
        ],
    )
    def sc_gather(table_hbm, idx_hbm, out_hbm, idx_v, rows_v, sems):
        wid = lax.axis_index("s") * 2 + lax.axis_index("c")
        base = wid * b_per_w
        pltpu.sync_copy(idx_hbm.at[wid], idx_v)
        pltpu.async_copy(table_hbm.at[idx_v.at[0]], rows_v.at[0], sems.at[0])
        pltpu.async_copy(table_hbm.at[idx_v.at[1]], rows_v.at[1], sems.at[1])

        def body(c, _):
            nxt = c + 2

            @pl.when(nxt < nch)
            def _():
                pltpu.async_copy(
                    table_hbm.at[idx_v.at[nxt]], rows_v.at[nxt % 3], sems.at[nxt % 3]
                )

            pltpu.make_async_copy(
                table_hbm.at[idx_v.at[c]], rows_v.at[c % 3], sems.at[c % 3]
            ).wait()
            pltpu.sync_copy(rows_v.at[c % 3], out_hbm.at[pl.ds(base + c * _CH, _CH)])
            return 0

        lax.fori_loop(0, nch, body, 0)

    return sc_gather


def _ln(h, g, b):
    mu = jnp.mean(h, axis=-1, keepdims=True)
    d = h - mu
    var = jnp.mean(d * d, axis=-1, keepdims=True)
    return d / jnp.sqrt(var + 1e-5) * g + b


def _gelu(h):
    return 0.5 * h * (1.0 + lax.erf(h * (1.0 / math.sqrt(2.0))))


def _mlp_body(g_ref, posd_ref, w1p_ref, b1_ref, g1_ref, be1_ref,
              w2_ref, b2_ref, g2_ref, be2_ref, out_ref, pdw_ref, mx_ref):
    g = pl.program_id(0)

    @pl.when(g == 0)
    def _():
        pdw_ref[:] = jnp.dot(
            posd_ref[:], w1p_ref[:], preferred_element_type=jnp.float32) - b1_ref[:]

    h = g_ref[:] - pdw_ref[:]
    h = _gelu(_ln(h, g1_ref[:], be1_ref[:]))
    h = jnp.dot(h, w2_ref[:], preferred_element_type=jnp.float32) + b2_ref[:]
    h = _gelu(_ln(h, g2_ref[:], be2_ref[:]))

    @pl.when(g == 0)
    def _():
        mx_ref[:] = h

    @pl.when(g > 0)
    def _():
        mx_ref[:] = jnp.maximum(mx_ref[:], h)

    @pl.when(g == _K - 1)
    def _():
        out_ref[:] = mx_ref[:]


def _mlp_call(gathered, posd_pad, w1pos, b1, g1, be1, w2, b2, g2, be2):
    full = lambda r, c: pl.BlockSpec((r, c), lambda g: (0, 0))
    return pl.pallas_call(
        _mlp_body,
        grid=(_K,),
        in_specs=[
            pl.BlockSpec((_M, _COUT), lambda g: (g, 0)),
            full(_M, 8),
            full(8, _COUT),
            full(1, _COUT),
            full(1, _COUT),
            full(1, _COUT),
            full(_COUT, _COUT),
            full(1, _COUT),
            full(1, _COUT),
            full(1, _COUT),
        ],
        out_specs=pl.BlockSpec((_M, _COUT), lambda g: (0, 0)),
        out_shape=jax.ShapeDtypeStruct((_M, _COUT), jnp.float32),
        scratch_shapes=[
            pltpu.VMEM((_M, _COUT), jnp.float32),
            pltpu.VMEM((_M, _COUT), jnp.float32),
        ],
    )(gathered, posd_pad, w1pos, b1, g1, be1, w2, b2, g2, be2)


def kernel(x, pos, batch, W1, b1, g1, beta1, W2, b2, g2, beta2):
    sidx = _sampled_idx()
    pos_down = pos[sidx]
    batch_down = batch[sidx]

    pd2 = jnp.sum(pos_down * pos_down, axis=1).reshape(_M, 1)
    pn2 = jnp.sum(pos * pos, axis=1).reshape(1, _N)
    z1 = jnp.zeros((_M, 1), jnp.float32)
    posd_pad = jnp.concatenate([pos_down, z1, pd2, z1, z1, z1], axis=1)
    zr = jnp.zeros((1, _N), jnp.float32)
    pos_t8 = jnp.concatenate([pos.T, zr, zr, pn2, zr, zr], axis=0)

    knn_idx, ups2d = _topk_call(pos_t8, posd_pad)
    upsample_idx = ups2d.reshape(_N)

    idx_kmajor = knn_idx.T.reshape(_NW, _M * _K // _NW // _CH, _CH)

    xp = jnp.concatenate([x, pos, jnp.zeros((_N, _DPAD - _CIN - 3), jnp.float32)], axis=1)
    w1cat = jnp.concatenate([W1, jnp.zeros((_DPAD - _CIN - 3, _COUT), jnp.float32)], axis=0)
    table = _premul_call(xp, w1cat)
    gathered = _make_sc_gather()(table, idx_kmajor)

    w1pos = jnp.concatenate([W1[_CIN:_CIN + 3], jnp.zeros((5, _COUT), jnp.float32)], axis=0)
    r = lambda v: v.reshape(1, _COUT)

    x_down = _mlp_call(gathered, posd_pad, w1pos,
                       r(b1), r(g1), r(beta1), W2, r(b2), r(g2), r(beta2))
    return (x_down, pos_down, batch_down, upsample_idx)
